# baseline (speedup 1.0000x reference)
import jax
import jax.numpy as jnp
from jax import lax
from jax.experimental import pallas as pl
from jax.experimental.pallas import tpu as pltpu
from jax.experimental.pallas import tpu_sc as plsc

B = 1024
C = 1000
T = 6
GAMMA = 0.2
BIG = 1 << 20

NC, NS, L = 2, 16, 16
NW = NC * NS
ROWS_W = B // NW
NCOL = 7


def _sc_body(x_hbm, y_hbm, out_hbm, x_v, y_v, o_v):
    wid = lax.axis_index("s") * NC + lax.axis_index("c")
    base = wid * ROWS_W
    pltpu.sync_copy(x_hbm.at[pl.ds(base * C, ROWS_W * C)], x_v)
    pltpu.sync_copy(y_hbm.at[pl.ds(base * T, ROWS_W * T)], y_v)

    lanes = jnp.arange(L, dtype=jnp.int32)
    for g in range(ROWS_W // L):
        rl = lanes + g * L

        yv = [plsc.load_gather(y_v, [rl * T + k]) for k in range(T)]
        xv = [plsc.load_gather(x_v, [rl * C + yv[k]]) for k in range(T)]

        mval = xv[1]
        rel = yv[1]
        for k in range(2, T):
            take = xv[k] < mval
            mval = jnp.where(take, xv[k], mval)
            rel = jnp.where(take, yv[k], rel)

        didx = []
        dval = []
        neg_inf = jnp.full((L,), -jnp.inf, jnp.float32)
        big = jnp.full((L,), BIG, jnp.int32)
        for k in range(1, T):
            drop = yv[k] != rel
            for j in range(1, k):
                drop = drop & (yv[j] != yv[k])
            didx.append(jnp.where(drop, yv[k], big))
            dval.append(jnp.where(drop, xv[k], neg_inf))

        jstar = rel
        for _ in range(T - 1):
            cnt = (didx[0] <= jstar).astype(jnp.int32)
            for k in range(1, T - 1):
                cnt = cnt + (didx[k] <= jstar).astype(jnp.int32)
            jstar = rel + cnt
        tj = plsc.load_gather(x_v, [rl * C + jstar])

        plsc.store_scatter(o_v, [rl], xv[0])
        plsc.store_scatter(o_v, [ROWS_W + rl], tj)
        for k in range(T - 1):
            plsc.store_scatter(o_v, [(2 + k) * ROWS_W + rl], dval[k])

    for c in range(NCOL):
        pltpu.sync_copy(
            o_v.at[pl.ds(c * ROWS_W, ROWS_W)],
            out_hbm.at[pl.ds(c * B + base, ROWS_W)],
        )


def _sc_gather(x, y):
    mesh = plsc.VectorSubcoreMesh(core_axis_name="c", subcore_axis_name="s")
    run = pl.kernel(
        _sc_body,
        mesh=mesh,
        out_type=jax.ShapeDtypeStruct((NCOL * B,), jnp.float32),
        scratch_types=[
            pltpu.VMEM((ROWS_W * C,), jnp.float32),
            pltpu.VMEM((ROWS_W * T,), jnp.int32),
            pltpu.VMEM((NCOL * ROWS_W,), jnp.float32),
        ],
        compiler_params=pltpu.CompilerParams(needs_layout_passes=False),
    )
    return run(x.reshape(-1), y.reshape(-1))


BLK = 128


def _tc_dense_body(x_ref, m_ref, s_ref):
    xb = x_ref[...]
    m = jnp.max(xb, axis=1)
    s = jnp.sum(jnp.exp(xb - m[:, None]), axis=1)
    m_ref[...] = m
    s_ref[...] = s


def _tc_dense(x):
    return pl.pallas_call(
        _tc_dense_body,
        grid=(B // BLK,),
        in_specs=[pl.BlockSpec((BLK, C), lambda i: (i, 0))],
        out_specs=[
            pl.BlockSpec((BLK,), lambda i: (i,)),
            pl.BlockSpec((BLK,), lambda i: (i,)),
        ],
        out_shape=[
            jax.ShapeDtypeStruct((B,), jnp.float32),
            jax.ShapeDtypeStruct((B,), jnp.float32),
        ],
    )(x)


def _tc_comb_body(g_ref, m_ref, s_ref, o_ref):
    m = m_ref[...]
    s = s_ref[...]
    g = g_ref[...]
    t0 = g[0, :]
    tj = g[1, :]
    d = g[2:7, :]
    c = jnp.sum(jnp.exp(d - m[None, :]), axis=0)
    lse_f = m + jnp.log(s)
    lse_m = m + jnp.log(s - c)
    p1 = jnp.sum(lse_f - t0)
    p2 = jnp.sum(lse_m - tj)
    o_ref[0, 0] = p1 / B + GAMMA * p2 / (B + 1e-8)


def _tc_comb(g, m, s):
    return pl.pallas_call(
        _tc_comb_body,
        out_specs=pl.BlockSpec(memory_space=pltpu.SMEM),
        out_shape=jax.ShapeDtypeStruct((1, 1), jnp.float32),
    )(g, m, s)


def kernel(x, y):
    scg = _sc_gather(x, y.astype(jnp.int32))
    m, s = _tc_dense(x)
    out = _tc_comb(scg.reshape(NCOL, B), m, s)
    return out[0, 0]


# speedup vs baseline: 1.1760x; 1.1760x over previous
import jax
import jax.numpy as jnp
from jax import lax
from jax.experimental import pallas as pl
from jax.experimental.pallas import tpu as pltpu
from jax.experimental.pallas import tpu_sc as plsc

B = 1024
C = 1000
T = 6
GAMMA = 0.2
BIG = 1 << 20

NC, NS, L = 2, 16, 16
NW = NC * NS
ROWS_W = B // NW
NCOL = 7


def _sc_body(x_hbm, y_hbm, out_hbm, x_v, y_v, o_v):
    wid = lax.axis_index("s") * NC + lax.axis_index("c")
    base = wid * ROWS_W
    pltpu.sync_copy(x_hbm.at[pl.ds(base, ROWS_W)], x_v)
    pltpu.sync_copy(y_hbm.at[pl.ds(base, ROWS_W)], y_v)

    lanes = jnp.arange(L, dtype=jnp.int32)
    for g in range(ROWS_W // L):
        rl = lanes + g * L

        yv = [
            plsc.load_gather(y_v, [rl, jnp.full((L,), k, jnp.int32)])
            for k in range(T)
        ]
        xv = [plsc.load_gather(x_v, [rl, yv[k]]) for k in range(T)]

        mval = xv[1]
        rel = yv[1]
        for k in range(2, T):
            take = xv[k] < mval
            mval = jnp.where(take, xv[k], mval)
            rel = jnp.where(take, yv[k], rel)

        didx = []
        dval = []
        neg_inf = jnp.full((L,), -jnp.inf, jnp.float32)
        big = jnp.full((L,), BIG, jnp.int32)
        for k in range(1, T):
            drop = yv[k] != rel
            for j in range(1, k):
                drop = drop & (yv[j] != yv[k])
            didx.append(jnp.where(drop, yv[k], big))
            dval.append(jnp.where(drop, xv[k], neg_inf))

        jstar = rel
        for _ in range(T - 1):
            cnt = (didx[0] <= jstar).astype(jnp.int32)
            for k in range(1, T - 1):
                cnt = cnt + (didx[k] <= jstar).astype(jnp.int32)
            jstar = rel + cnt
        tj = plsc.load_gather(x_v, [rl, jstar])

        plsc.store_scatter(o_v, [rl], xv[0])
        plsc.store_scatter(o_v, [ROWS_W + rl], tj)
        for k in range(T - 1):
            plsc.store_scatter(o_v, [(2 + k) * ROWS_W + rl], dval[k])

    for c in range(NCOL):
        pltpu.sync_copy(
            o_v.at[pl.ds(c * ROWS_W, ROWS_W)],
            out_hbm.at[c, pl.ds(base, ROWS_W)],
        )


def _sc_gather(x, y):
    mesh = plsc.VectorSubcoreMesh(core_axis_name="c", subcore_axis_name="s")
    run = pl.kernel(
        _sc_body,
        mesh=mesh,
        out_type=jax.ShapeDtypeStruct((8, B), jnp.float32),
        scratch_types=[
            pltpu.VMEM((ROWS_W, C), jnp.float32),
            pltpu.VMEM((ROWS_W, T), jnp.int32),
            pltpu.VMEM((8 * ROWS_W,), jnp.float32),
        ],
        compiler_params=pltpu.CompilerParams(needs_layout_passes=False),
    )
    return run(x, y)


BLK = 128


def _tc_dense_body(x_ref, m_ref, s_ref):
    xb = x_ref[...]
    m = jnp.max(xb, axis=1)
    s = jnp.sum(jnp.exp(xb - m[:, None]), axis=1)
    m_ref[...] = m
    s_ref[...] = s


def _tc_dense(x):
    return pl.pallas_call(
        _tc_dense_body,
        grid=(B // BLK,),
        in_specs=[pl.BlockSpec((BLK, C), lambda i: (i, 0))],
        out_specs=[
            pl.BlockSpec((BLK,), lambda i: (i,)),
            pl.BlockSpec((BLK,), lambda i: (i,)),
        ],
        out_shape=[
            jax.ShapeDtypeStruct((B,), jnp.float32),
            jax.ShapeDtypeStruct((B,), jnp.float32),
        ],
    )(x)


def _tc_comb_body(g_ref, m_ref, s_ref, o_ref):
    m = m_ref[...]
    s = s_ref[...]
    g = g_ref[...]
    t0 = g[0, :]
    tj = g[1, :]
    d = g[2:7, :]
    c = jnp.sum(jnp.exp(d - m[None, :]), axis=0)
    lse_f = m + jnp.log(s)
    lse_m = m + jnp.log(s - c)
    p1 = jnp.sum(lse_f - t0)
    p2 = jnp.sum(lse_m - tj)
    o_ref[0, 0] = p1 / B + GAMMA * p2 / (B + 1e-8)


def _tc_comb(g, m, s):
    return pl.pallas_call(
        _tc_comb_body,
        out_specs=pl.BlockSpec(memory_space=pltpu.SMEM),
        out_shape=jax.ShapeDtypeStruct((1, 1), jnp.float32),
    )(g, m, s)


def kernel(x, y):
    scg = _sc_gather(x, y.astype(jnp.int32))
    m, s = _tc_dense(x)
    out = _tc_comb(scg, m, s)
    return out[0, 0]


# speedup vs baseline: 1.1779x; 1.0016x over previous
import jax
import jax.numpy as jnp
from jax import lax
from jax.experimental import pallas as pl
from jax.experimental.pallas import tpu as pltpu
from jax.experimental.pallas import tpu_sc as plsc

B = 1024
C = 1000
T = 6
GAMMA = 0.2
BIG = 1 << 20

NC, NS, L = 2, 16, 16
NW = NC * NS
ROWS_W = B // NW
NCOL = 7


def _sc_body(x_hbm, y_hbm, out_hbm, x_v, y_v, o_v):
    wid = lax.axis_index("s") * NC + lax.axis_index("c")
    base = wid * ROWS_W
    pltpu.sync_copy(x_hbm.at[pl.ds(base, ROWS_W)], x_v)
    pltpu.sync_copy(y_hbm.at[pl.ds(base, ROWS_W)], y_v)

    lanes = jnp.arange(L, dtype=jnp.int32)
    for g in range(ROWS_W // L):
        rl = lanes + g * L

        yv = [
            plsc.load_gather(y_v, [rl, jnp.full((L,), k, jnp.int32)])
            for k in range(T)
        ]
        xv = [plsc.load_gather(x_v, [rl, yv[k]]) for k in range(T)]

        mval = xv[1]
        rel = yv[1]
        for k in range(2, T):
            take = xv[k] < mval
            mval = jnp.where(take, xv[k], mval)
            rel = jnp.where(take, yv[k], rel)

        didx = []
        dval = []
        neg_inf = jnp.full((L,), -jnp.inf, jnp.float32)
        big = jnp.full((L,), BIG, jnp.int32)
        for k in range(1, T):
            drop = yv[k] != rel
            for j in range(1, k):
                drop = drop & (yv[j] != yv[k])
            didx.append(jnp.where(drop, yv[k], big))
            dval.append(jnp.where(drop, xv[k], neg_inf))

        jstar = rel
        for _ in range(T - 1):
            cnt = (didx[0] <= jstar).astype(jnp.int32)
            for k in range(1, T - 1):
                cnt = cnt + (didx[k] <= jstar).astype(jnp.int32)
            jstar = rel + cnt
        tj = plsc.load_gather(x_v, [rl, jstar])

        plsc.store_scatter(o_v, [rl], xv[0])
        plsc.store_scatter(o_v, [ROWS_W + rl], tj)
        for k in range(T - 1):
            plsc.store_scatter(o_v, [(2 + k) * ROWS_W + rl], dval[k])

    for c in range(NCOL):
        pltpu.sync_copy(
            o_v.at[pl.ds(c * ROWS_W, ROWS_W)],
            out_hbm.at[c, pl.ds(base, ROWS_W)],
        )


def _sc_gather(x, y):
    mesh = plsc.VectorSubcoreMesh(core_axis_name="c", subcore_axis_name="s")
    run = pl.kernel(
        _sc_body,
        mesh=mesh,
        out_type=jax.ShapeDtypeStruct((8, B), jnp.float32),
        scratch_types=[
            pltpu.VMEM((ROWS_W, C), jnp.float32),
            pltpu.VMEM((ROWS_W, T), jnp.int32),
            pltpu.VMEM((8 * ROWS_W,), jnp.float32),
        ],
        compiler_params=pltpu.CompilerParams(
            needs_layout_passes=False, use_tc_tiling_on_sc=True
        ),
    )
    return run(x, y)


BLK = 128


def _tc_dense_body(x_ref, m_ref, s_ref):
    xb = x_ref[...]
    m = jnp.max(xb, axis=1)
    s = jnp.sum(jnp.exp(xb - m[:, None]), axis=1)
    m_ref[...] = m
    s_ref[...] = s


def _tc_dense(x):
    return pl.pallas_call(
        _tc_dense_body,
        grid=(B // BLK,),
        in_specs=[pl.BlockSpec((BLK, C), lambda i: (i, 0))],
        out_specs=[
            pl.BlockSpec((BLK,), lambda i: (i,)),
            pl.BlockSpec((BLK,), lambda i: (i,)),
        ],
        out_shape=[
            jax.ShapeDtypeStruct((B,), jnp.float32),
            jax.ShapeDtypeStruct((B,), jnp.float32),
        ],
    )(x)


def _tc_comb_body(g_ref, m_ref, s_ref, o_ref):
    m = m_ref[...]
    s = s_ref[...]
    g = g_ref[...]
    t0 = g[0, :]
    tj = g[1, :]
    d = g[2:7, :]
    c = jnp.sum(jnp.exp(d - m[None, :]), axis=0)
    lse_f = m + jnp.log(s)
    lse_m = m + jnp.log(s - c)
    p1 = jnp.sum(lse_f - t0)
    p2 = jnp.sum(lse_m - tj)
    o_ref[0, 0] = p1 / B + GAMMA * p2 / (B + 1e-8)


def _tc_comb(g, m, s):
    return pl.pallas_call(
        _tc_comb_body,
        out_specs=pl.BlockSpec(memory_space=pltpu.SMEM),
        out_shape=jax.ShapeDtypeStruct((1, 1), jnp.float32),
    )(g, m, s)


def kernel(x, y):
    scg = _sc_gather(x, y.astype(jnp.int32))
    m, s = _tc_dense(x)
    out = _tc_comb(scg, m, s)
    return out[0, 0]


# speedup vs baseline: 1.2412x; 1.0537x over previous
import jax
import jax.numpy as jnp
from jax import lax
from jax.experimental import pallas as pl
from jax.experimental.pallas import tpu as pltpu
from jax.experimental.pallas import tpu_sc as plsc

B = 1024
C = 1000
T = 6
GAMMA = 0.2
BIG = 1 << 20

NC, NS, L = 2, 16, 16
NW = NC * NS
ROWS_W = B // NW
NCOL = 7


NSLAB = 8
SLAB = B // NSLAB


def _sc_body(xt_hbm, yt_hbm, out_hbm, x_v, y_v, o_v):
    wid = lax.axis_index("s") * NC + lax.axis_index("c")

    @pl.when(wid < NSLAB)
    def _():
        base = wid * SLAB
        pltpu.sync_copy(xt_hbm.at[:, pl.ds(base, SLAB)], x_v)
        pltpu.sync_copy(yt_hbm.at[:, pl.ds(base, SLAB)], y_v)

        lanes = jnp.arange(L, dtype=jnp.int32)
        for g in range(SLAB // L):
            rl = lanes + g * L

            yv = [
                plsc.load_gather(y_v, [jnp.full((L,), k, jnp.int32), rl])
                for k in range(T)
            ]
            xv = [plsc.load_gather(x_v, [yv[k], rl]) for k in range(T)]

            mval = xv[1]
            rel = yv[1]
            for k in range(2, T):
                take = xv[k] < mval
                mval = jnp.where(take, xv[k], mval)
                rel = jnp.where(take, yv[k], rel)

            didx = []
            dval = []
            neg_inf = jnp.full((L,), -jnp.inf, jnp.float32)
            big = jnp.full((L,), BIG, jnp.int32)
            for k in range(1, T):
                drop = yv[k] != rel
                for j in range(1, k):
                    drop = drop & (yv[j] != yv[k])
                didx.append(jnp.where(drop, yv[k], big))
                dval.append(jnp.where(drop, xv[k], neg_inf))

            jstar = rel
            for _ in range(T - 1):
                cnt = (didx[0] <= jstar).astype(jnp.int32)
                for k in range(1, T - 1):
                    cnt = cnt + (didx[k] <= jstar).astype(jnp.int32)
                jstar = rel + cnt
            tj = plsc.load_gather(x_v, [jstar, rl])

            plsc.store_scatter(o_v, [rl], xv[0])
            plsc.store_scatter(o_v, [SLAB + rl], tj)
            for k in range(T - 1):
                plsc.store_scatter(o_v, [(2 + k) * SLAB + rl], dval[k])

        for c in range(NCOL):
            pltpu.sync_copy(
                o_v.at[pl.ds(c * SLAB, SLAB)],
                out_hbm.at[c, pl.ds(base, SLAB)],
            )


def _sc_gather(xt, yt):
    mesh = plsc.VectorSubcoreMesh(core_axis_name="c", subcore_axis_name="s")
    run = pl.kernel(
        _sc_body,
        mesh=mesh,
        out_type=jax.ShapeDtypeStruct((8, B), jnp.float32),
        scratch_types=[
            pltpu.VMEM((C, SLAB), jnp.float32),
            pltpu.VMEM((T, SLAB), jnp.int32),
            pltpu.VMEM((8 * SLAB,), jnp.float32),
        ],
        compiler_params=pltpu.CompilerParams(
            needs_layout_passes=False, use_tc_tiling_on_sc=True
        ),
    )
    return run(xt, yt)


BLK = 128


def _tc_dense_body(x_ref, m_ref, s_ref):
    xb = x_ref[...]
    m = jnp.max(xb, axis=0)
    s = jnp.sum(jnp.exp(xb - m[None, :]), axis=0)
    m_ref[...] = m
    s_ref[...] = s


def _tc_dense(xt):
    return pl.pallas_call(
        _tc_dense_body,
        grid=(B // BLK,),
        in_specs=[pl.BlockSpec((C, BLK), lambda i: (0, i))],
        out_specs=[
            pl.BlockSpec((BLK,), lambda i: (i,)),
            pl.BlockSpec((BLK,), lambda i: (i,)),
        ],
        out_shape=[
            jax.ShapeDtypeStruct((B,), jnp.float32),
            jax.ShapeDtypeStruct((B,), jnp.float32),
        ],
    )(xt)


def _tc_comb_body(g_ref, m_ref, s_ref, o_ref):
    m = m_ref[...]
    s = s_ref[...]
    g = g_ref[...]
    t0 = g[0, :]
    tj = g[1, :]
    d = g[2:7, :]
    c = jnp.sum(jnp.exp(d - m[None, :]), axis=0)
    lse_f = m + jnp.log(s)
    lse_m = m + jnp.log(s - c)
    p1 = jnp.sum(lse_f - t0)
    p2 = jnp.sum(lse_m - tj)
    o_ref[0, 0] = p1 / B + GAMMA * p2 / (B + 1e-8)


def _tc_comb(g, m, s):
    return pl.pallas_call(
        _tc_comb_body,
        out_specs=pl.BlockSpec(memory_space=pltpu.SMEM),
        out_shape=jax.ShapeDtypeStruct((1, 1), jnp.float32),
    )(g, m, s)


def kernel(x, y):
    xt = x.T
    yt = y.astype(jnp.int32).T
    scg = _sc_gather(xt, yt)
    m, s = _tc_dense(xt)
    out = _tc_comb(scg, m, s)
    return out[0, 0]


# speedup vs baseline: 1.3050x; 1.0514x over previous
import jax
import jax.numpy as jnp
from jax import lax
from jax.experimental import pallas as pl
from jax.experimental.pallas import tpu as pltpu
from jax.experimental.pallas import tpu_sc as plsc

B = 1024
C = 1000
T = 6
GAMMA = 0.2
BIG = 1 << 20

NC, NS, L = 2, 16, 16
NW = NC * NS
ROWS_W = B // NW
NCOL = 7


NSLAB = 8
SLAB = B // NSLAB


def _sc_body(xt_hbm, yt_hbm, out_hbm, x_v, y_v, o_v, sem):
    wid = lax.axis_index("s") * NC + lax.axis_index("c")

    @pl.when(wid < NSLAB)
    def _():
        base = wid * SLAB
        xcp = pltpu.make_async_copy(xt_hbm.at[:, pl.ds(base, SLAB)], x_v, sem)
        xcp.start()
        pltpu.sync_copy(yt_hbm.at[:, pl.ds(base, SLAB)], y_v)
        xcp.wait()

        lanes = jnp.arange(L, dtype=jnp.int32)

        def group(g, carry):
            rl = lanes + g * L

            yv = [
                plsc.load_gather(y_v, [jnp.full((L,), k, jnp.int32), rl])
                for k in range(T)
            ]
            xv = [plsc.load_gather(x_v, [yv[k], rl]) for k in range(T)]

            mval = xv[1]
            rel = yv[1]
            for k in range(2, T):
                take = xv[k] < mval
                mval = jnp.where(take, xv[k], mval)
                rel = jnp.where(take, yv[k], rel)

            didx = []
            dval = []
            neg_inf = jnp.full((L,), -jnp.inf, jnp.float32)
            big = jnp.full((L,), BIG, jnp.int32)
            for k in range(1, T):
                drop = yv[k] != rel
                for j in range(1, k):
                    drop = drop & (yv[j] != yv[k])
                didx.append(jnp.where(drop, yv[k], big))
                dval.append(jnp.where(drop, xv[k], neg_inf))

            jstar = rel
            for _ in range(T - 1):
                cnt = (didx[0] <= jstar).astype(jnp.int32)
                for k in range(1, T - 1):
                    cnt = cnt + (didx[k] <= jstar).astype(jnp.int32)
                jstar = rel + cnt
            tj = plsc.load_gather(x_v, [jstar, rl])

            plsc.store_scatter(o_v, [rl], xv[0])
            plsc.store_scatter(o_v, [SLAB + rl], tj)
            for k in range(T - 1):
                plsc.store_scatter(o_v, [(2 + k) * SLAB + rl], dval[k])
            return carry

        lax.fori_loop(0, SLAB // L, group, 0)

        for c in range(NCOL):
            pltpu.sync_copy(
                o_v.at[pl.ds(c * SLAB, SLAB)],
                out_hbm.at[c, pl.ds(base, SLAB)],
            )


def _sc_gather(xt, yt):
    mesh = plsc.VectorSubcoreMesh(core_axis_name="c", subcore_axis_name="s")
    run = pl.kernel(
        _sc_body,
        mesh=mesh,
        out_type=jax.ShapeDtypeStruct((8, B), jnp.float32),
        scratch_types=[
            pltpu.VMEM((C, SLAB), jnp.float32),
            pltpu.VMEM((T, SLAB), jnp.int32),
            pltpu.VMEM((8 * SLAB,), jnp.float32),
            pltpu.SemaphoreType.DMA,
        ],
        compiler_params=pltpu.CompilerParams(
            needs_layout_passes=False, use_tc_tiling_on_sc=True
        ),
    )
    return run(xt, yt)


BLK = 128


def _tc_dense_body(x_ref, m_ref, s_ref):
    xb = x_ref[...]
    m = jnp.max(xb, axis=0)
    s = jnp.sum(jnp.exp(xb - m[None, :]), axis=0)
    m_ref[...] = m
    s_ref[...] = s


def _tc_dense(xt):
    return pl.pallas_call(
        _tc_dense_body,
        grid=(B // BLK,),
        in_specs=[pl.BlockSpec((C, BLK), lambda i: (0, i))],
        out_specs=[
            pl.BlockSpec((BLK,), lambda i: (i,)),
            pl.BlockSpec((BLK,), lambda i: (i,)),
        ],
        out_shape=[
            jax.ShapeDtypeStruct((B,), jnp.float32),
            jax.ShapeDtypeStruct((B,), jnp.float32),
        ],
    )(xt)


def _tc_comb_body(g_ref, m_ref, s_ref, o_ref):
    m = m_ref[...]
    s = s_ref[...]
    g = g_ref[...]
    t0 = g[0, :]
    tj = g[1, :]
    d = g[2:7, :]
    c = jnp.sum(jnp.exp(d - m[None, :]), axis=0)
    lse_f = m + jnp.log(s)
    lse_m = m + jnp.log(s - c)
    p1 = jnp.sum(lse_f - t0)
    p2 = jnp.sum(lse_m - tj)
    o_ref[0, 0] = p1 / B + GAMMA * p2 / (B + 1e-8)


def _tc_comb(g, m, s):
    return pl.pallas_call(
        _tc_comb_body,
        out_specs=pl.BlockSpec(memory_space=pltpu.SMEM),
        out_shape=jax.ShapeDtypeStruct((1, 1), jnp.float32),
    )(g, m, s)


def kernel(x, y):
    xt = x.T
    yt = y.astype(jnp.int32).T
    scg = _sc_gather(xt, yt)
    m, s = _tc_dense(xt)
    out = _tc_comb(scg, m, s)
    return out[0, 0]


# speedup vs baseline: 1.3116x; 1.0051x over previous
import jax
import jax.numpy as jnp
from jax import lax
from jax.experimental import pallas as pl
from jax.experimental.pallas import tpu as pltpu
from jax.experimental.pallas import tpu_sc as plsc

B = 1024
C = 1000
T = 6
GAMMA = 0.2
BIG = 1 << 20

NC, NS, L = 2, 16, 16
NW = NC * NS
ROWS_W = B // NW
NCOL = 7


NSLAB = 8
SLAB = B // NSLAB


def _sc_body(xt_hbm, yt_hbm, out_hbm, x_v, y_v, o_v, sem):
    wid = lax.axis_index("s") * NC + lax.axis_index("c")

    @pl.when(wid < NSLAB)
    def _():
        base = wid * SLAB
        xcp = pltpu.make_async_copy(xt_hbm.at[:, pl.ds(base, SLAB)], x_v, sem)
        xcp.start()
        pltpu.sync_copy(yt_hbm.at[:, pl.ds(base, SLAB)], y_v)
        xcp.wait()

        lanes = jnp.arange(L, dtype=jnp.int32)

        def group(g, carry):
            gl = g * L
            rl = lanes + gl

            yv = [y_v[k, pl.ds(gl, L)] for k in range(T)]
            xv = [plsc.load_gather(x_v, [yv[k], rl]) for k in range(T)]

            mval = xv[1]
            rel = yv[1]
            for k in range(2, T):
                take = xv[k] < mval
                mval = jnp.where(take, xv[k], mval)
                rel = jnp.where(take, yv[k], rel)

            didx = []
            dval = []
            neg_inf = jnp.full((L,), -jnp.inf, jnp.float32)
            big = jnp.full((L,), BIG, jnp.int32)
            for k in range(1, T):
                drop = yv[k] != rel
                for j in range(1, k):
                    drop = drop & (yv[j] != yv[k])
                didx.append(jnp.where(drop, yv[k], big))
                dval.append(jnp.where(drop, xv[k], neg_inf))

            jstar = rel
            for _ in range(T - 1):
                cnt = (didx[0] <= jstar).astype(jnp.int32)
                for k in range(1, T - 1):
                    cnt = cnt + (didx[k] <= jstar).astype(jnp.int32)
                jstar = rel + cnt
            tj = plsc.load_gather(x_v, [jstar, rl])

            o_v[pl.ds(gl, L)] = xv[0]
            o_v[pl.ds(SLAB + gl, L)] = tj
            for k in range(T - 1):
                o_v[pl.ds((2 + k) * SLAB + gl, L)] = dval[k]
            return carry

        lax.fori_loop(0, SLAB // L, group, 0)

        for c in range(NCOL):
            pltpu.sync_copy(
                o_v.at[pl.ds(c * SLAB, SLAB)],
                out_hbm.at[c, pl.ds(base, SLAB)],
            )


def _sc_gather(xt, yt):
    mesh = plsc.VectorSubcoreMesh(core_axis_name="c", subcore_axis_name="s")
    run = pl.kernel(
        _sc_body,
        mesh=mesh,
        out_type=jax.ShapeDtypeStruct((8, B), jnp.float32),
        scratch_types=[
            pltpu.VMEM((C, SLAB), jnp.float32),
            pltpu.VMEM((T, SLAB), jnp.int32),
            pltpu.VMEM((8 * SLAB,), jnp.float32),
            pltpu.SemaphoreType.DMA,
        ],
        compiler_params=pltpu.CompilerParams(
            needs_layout_passes=False, use_tc_tiling_on_sc=True
        ),
    )
    return run(xt, yt)


BLK = 128


def _tc_dense_body(x_ref, m_ref, s_ref):
    xb = x_ref[...]
    m = jnp.max(xb, axis=0)
    s = jnp.sum(jnp.exp(xb - m[None, :]), axis=0)
    m_ref[...] = m
    s_ref[...] = s


def _tc_dense(xt):
    return pl.pallas_call(
        _tc_dense_body,
        grid=(B // BLK,),
        in_specs=[pl.BlockSpec((C, BLK), lambda i: (0, i))],
        out_specs=[
            pl.BlockSpec((BLK,), lambda i: (i,)),
            pl.BlockSpec((BLK,), lambda i: (i,)),
        ],
        out_shape=[
            jax.ShapeDtypeStruct((B,), jnp.float32),
            jax.ShapeDtypeStruct((B,), jnp.float32),
        ],
    )(xt)


def _tc_comb_body(g_ref, m_ref, s_ref, o_ref):
    m = m_ref[...]
    s = s_ref[...]
    g = g_ref[...]
    t0 = g[0, :]
    tj = g[1, :]
    d = g[2:7, :]
    c = jnp.sum(jnp.exp(d - m[None, :]), axis=0)
    lse_f = m + jnp.log(s)
    lse_m = m + jnp.log(s - c)
    p1 = jnp.sum(lse_f - t0)
    p2 = jnp.sum(lse_m - tj)
    o_ref[0, 0] = p1 / B + GAMMA * p2 / (B + 1e-8)


def _tc_comb(g, m, s):
    return pl.pallas_call(
        _tc_comb_body,
        out_specs=pl.BlockSpec(memory_space=pltpu.SMEM),
        out_shape=jax.ShapeDtypeStruct((1, 1), jnp.float32),
    )(g, m, s)


def kernel(x, y):
    xt = x.T
    yt = y.astype(jnp.int32).T
    scg = _sc_gather(xt, yt)
    m, s = _tc_dense(xt)
    out = _tc_comb(scg, m, s)
    return out[0, 0]
